# Initial kernel scaffold; baseline (speedup 1.0000x reference)
import jax
import jax.numpy as jnp
from jax.experimental import pallas as pl


def kernel(H, edge_index, edge_values, W0, b0, W1, b1):
    raise NotImplementedError("write your pallas kernel here")


# speedup vs baseline: 6.6922x; 6.6922x over previous
import functools

import jax
import jax.numpy as jnp
from jax import lax
from jax.experimental import pallas as pl
from jax.experimental.pallas import tpu as pltpu
from jax.experimental.pallas import tpu_sc as plsc

N_NODES = 10000
N_EDGES = 320000
D_FEAT = 128

NC = 2
NS = 16
LANES = 16

EPT = N_EDGES // (NC * NS)
CHUNK = 256
NCHUNK = EPT // CHUNK
TAIL = EPT - NCHUNK * CHUNK
RPT = 624
REM = N_NODES - RPT * NS


def _process_chunk(h_hbm, row_hbm, col_hbm, val_hbm, acc_sh, sem,
                   rows_v, rowi_v, coli_v, val_v, base, n):
  pltpu.sync_copy(row_hbm.at[pl.ds(base, n)], rowi_v)
  pltpu.sync_copy(col_hbm.at[pl.ds(base, n)], coli_v)
  pltpu.sync_copy(val_hbm.at[pl.ds(base, n)], val_v)
  pltpu.async_copy(h_hbm.at[coli_v], rows_v, sem).wait()

  dnums = lax.GatherDimensionNumbers(
      offset_dims=(), collapsed_slice_dims=(0,), start_index_map=(0,))

  def scale_group(g, _):
    val16 = val_v[pl.ds(g * LANES, LANES)]
    for i in range(LANES):
      vsplat = lax.gather(
          val16, jnp.full((LANES, 1), i, jnp.int32), dnums,
          slice_sizes=(1,),
          mode=lax.GatherScatterMode.PROMISE_IN_BOUNDS)
      e = g * LANES + i
      for j in range(D_FEAT // LANES):
        sl = (e, pl.ds(j * LANES, LANES))
        rows_v[sl] = rows_v[sl] * vsplat
    return 0
  lax.fori_loop(0, n // LANES, scale_group, 0)

  pltpu.sync_copy(rows_v, acc_sh.at[rowi_v], add=True)


def _spmm_body(h_hbm, row_hbm, col_hbm, val_hbm, p_hbm,
               rows_v, rowi_v, coli_v, val_v,
               rows_t, rowi_t, coli_t, val_t, acc_sh, sem):
  c = lax.axis_index("c")
  s = lax.axis_index("s")
  tile = c * NS + s
  ebase = tile * EPT

  def zero_body(r, _):
    for j in range(D_FEAT // LANES):
      rows_v[r, pl.ds(j * LANES, LANES)] = jnp.zeros((LANES,), jnp.float32)
    return 0
  lax.fori_loop(0, CHUNK, zero_body, 0)

  def zero_rows(start, cnt):
    done = 0
    while done < cnt:
      step = min(CHUNK, cnt - done)
      pltpu.sync_copy(rows_v.at[pl.ds(0, step)],
                      acc_sh.at[pl.ds(start + done, step)])
      done += step

  zero_rows(s * RPT, RPT)

  @pl.when(s == NS - 1)
  def _():
    zero_rows(NS * RPT, REM)

  plsc.subcore_barrier()

  def chunk_body(k, _):
    _process_chunk(h_hbm, row_hbm, col_hbm, val_hbm, acc_sh, sem,
                   rows_v, rowi_v, coli_v, val_v, ebase + k * CHUNK, CHUNK)
    return 0
  lax.fori_loop(0, NCHUNK, chunk_body, 0)
  if TAIL:
    _process_chunk(h_hbm, row_hbm, col_hbm, val_hbm, acc_sh, sem,
                   rows_t, rowi_t, coli_t, val_t, ebase + NCHUNK * CHUNK, TAIL)

  plsc.subcore_barrier()
  pltpu.sync_copy(acc_sh.at[pl.ds(s * RPT, RPT)],
                  p_hbm.at[c, pl.ds(s * RPT, RPT)])

  @pl.when(s == NS - 1)
  def _():
    pltpu.sync_copy(acc_sh.at[pl.ds(NS * RPT, REM)],
                    p_hbm.at[c, pl.ds(NS * RPT, REM)])


@jax.jit
def _spmm(h, row, col, val):
  mesh = plsc.VectorSubcoreMesh(core_axis_name="c", subcore_axis_name="s")
  return pl.kernel(
      _spmm_body,
      out_type=jax.ShapeDtypeStruct((NC, N_NODES, D_FEAT), jnp.float32),
      mesh=mesh,
      scratch_types=[
          pltpu.VMEM((CHUNK, D_FEAT), jnp.float32),
          pltpu.VMEM((CHUNK,), jnp.int32),
          pltpu.VMEM((CHUNK,), jnp.int32),
          pltpu.VMEM((CHUNK,), jnp.float32),
          pltpu.VMEM((TAIL, D_FEAT), jnp.float32),
          pltpu.VMEM((TAIL,), jnp.int32),
          pltpu.VMEM((TAIL,), jnp.int32),
          pltpu.VMEM((TAIL,), jnp.float32),
          pltpu.VMEM_SHARED((N_NODES, D_FEAT), jnp.float32),
          pltpu.SemaphoreType.DMA,
      ],
      name="gcn_spmm_sc",
  )(h, row, col, val)


def _dense_body(p_ref, w_ref, b_ref, o_ref):
  x = p_ref[0] + p_ref[1]
  y = jnp.dot(x, w_ref[...], preferred_element_type=jnp.float32) + b_ref[...]
  o_ref[...] = jnp.maximum(y, 0.0)


BLK = 1000


@jax.jit
def _dense(p, w, b):
  b2 = b.reshape(1, D_FEAT)
  grid = (N_NODES // BLK,)
  return pl.pallas_call(
      _dense_body,
      grid=grid,
      in_specs=[
          pl.BlockSpec((NC, BLK, D_FEAT), lambda i: (0, i, 0)),
          pl.BlockSpec((D_FEAT, D_FEAT), lambda i: (0, 0)),
          pl.BlockSpec((1, D_FEAT), lambda i: (0, 0)),
      ],
      out_specs=pl.BlockSpec((BLK, D_FEAT), lambda i: (i, 0)),
      out_shape=jax.ShapeDtypeStruct((N_NODES, D_FEAT), jnp.float32),
      name="gcn_dense_tc",
  )(p, w, b2)


def kernel(H, edge_index, edge_values, W0, b0, W1, b1):
  row = edge_index[0].astype(jnp.int32)
  col = edge_index[1].astype(jnp.int32)
  val = edge_values.astype(jnp.float32)
  p0 = _spmm(H, row, col, val)
  h1 = _dense(p0, W0, b0)
  p1 = _spmm(h1, row, col, val)
  h2 = _dense(p1, W1, b1)
  return h2


# speedup vs baseline: 12.7440x; 1.9043x over previous
import jax
import jax.numpy as jnp
from jax import lax
from jax.experimental import pallas as pl
from jax.experimental.pallas import tpu as pltpu
from jax.experimental.pallas import tpu_sc as plsc

N_NODES = 10000
N_EDGES = 320000
D_FEAT = 128

NC = 2
NS = 16
LANES = 16
NSLOT = 3

EPT = N_EDGES // (NC * NS)
CHUNK = 96
NCHUNK = EPT // CHUNK
TAIL = EPT - NCHUNK * CHUNK
RPT = 624
REM = N_NODES - RPT * NS

_DNUMS = lax.GatherDimensionNumbers(
    offset_dims=(), collapsed_slice_dims=(0,), start_index_map=(0,))


def _scale(rows_v, val_v, n):
  def scale_group(g, _):
    val16 = val_v[pl.ds(g * LANES, LANES)]
    for i in range(LANES):
      vsplat = lax.gather(
          val16, jnp.full((LANES, 1), i, jnp.int32), _DNUMS,
          slice_sizes=(1,), mode=lax.GatherScatterMode.PROMISE_IN_BOUNDS)
      e = g * LANES + i
      for j in range(D_FEAT // LANES):
        sl = (e, pl.ds(j * LANES, LANES))
        rows_v[sl] = rows_v[sl] * vsplat
    return 0
  lax.fori_loop(0, n // LANES, scale_group, 0)


def _spmm_body(h_hbm, row_hbm, col_hbm, val_hbm, p_hbm, *refs):
  rows = refs[0:3]
  rowi = refs[3:6]
  coli = refs[6:9]
  vals = refs[9:12]
  rowi_s = refs[12:15]
  rows_t, rowi_t, coli_t, val_t = refs[15:19]
  acc_sh = refs[19]
  gat_sem = refs[20:23]
  scat_sem = refs[23:26]
  idx_sem = refs[26:29]
  sem_t = refs[29]

  c = lax.axis_index("c")
  s = lax.axis_index("s")
  tile = c * NS + s
  ebase = tile * EPT

  def zero_body(r, _):
    for j in range(D_FEAT // LANES):
      rows[0][r, pl.ds(j * LANES, LANES)] = jnp.zeros((LANES,), jnp.float32)
    return 0
  lax.fori_loop(0, CHUNK, zero_body, 0)

  def zero_rows(start, cnt):
    done = 0
    while done < cnt:
      step = min(CHUNK, cnt - done)
      pltpu.sync_copy(rows[0].at[pl.ds(0, step)],
                      acc_sh.at[pl.ds(start + done, step)])
      done += step

  zero_rows(s * RPT, RPT)

  @pl.when(s == NS - 1)
  def _():
    zero_rows(NS * RPT, REM)

  plsc.subcore_barrier()

  def idx_start(k, sl):
    base = ebase + k * CHUNK
    pltpu.async_copy(row_hbm.at[pl.ds(base, CHUNK)], rowi[sl], idx_sem[sl])
    pltpu.async_copy(col_hbm.at[pl.ds(base, CHUNK)], coli[sl], idx_sem[sl])
    pltpu.async_copy(val_hbm.at[pl.ds(base, CHUNK)], vals[sl], idx_sem[sl])

  def idx_wait(k, sl):
    base = ebase + k * CHUNK
    pltpu.make_async_copy(
        row_hbm.at[pl.ds(base, CHUNK)], rowi[sl], idx_sem[sl]).wait()
    pltpu.make_async_copy(
        col_hbm.at[pl.ds(base, CHUNK)], coli[sl], idx_sem[sl]).wait()
    pltpu.make_async_copy(
        val_hbm.at[pl.ds(base, CHUNK)], vals[sl], idx_sem[sl]).wait()

  def scat_wait(sl):
    pltpu.make_async_copy(rows[sl], acc_sh.at[rowi_s[sl]],
                          scat_sem[sl]).wait()

  def iteration(k, sl, sl1, sl2):
    @pl.when(jnp.logical_and(k >= 2, k + 1 < NCHUNK))
    def _():
      scat_wait(sl1)

    @pl.when(k + 1 < NCHUNK)
    def _():
      idx_wait(k + 1, sl1)
      pltpu.async_copy(h_hbm.at[coli[sl1]], rows[sl1], gat_sem[sl1])

    @pl.when(k + 2 < NCHUNK)
    def _():
      idx_start(k + 2, sl2)

    pltpu.make_async_copy(h_hbm.at[coli[sl]], rows[sl], gat_sem[sl]).wait()
    for g in range(CHUNK // LANES):
      rowi_s[sl][pl.ds(g * LANES, LANES)] = rowi[sl][pl.ds(g * LANES, LANES)]
    _scale(rows[sl], vals[sl], CHUNK)
    pltpu.async_copy(rows[sl], acc_sh.at[rowi_s[sl]], scat_sem[sl],
                     add=True)

  idx_start(0, 0)
  idx_start(1, 1)
  idx_wait(0, 0)
  pltpu.async_copy(h_hbm.at[coli[0]], rows[0], gat_sem[0])

  def triple(t, _):
    k = t * NSLOT
    iteration(k, 0, 1, 2)
    iteration(k + 1, 1, 2, 0)
    iteration(k + 2, 2, 0, 1)
    return 0
  assert NCHUNK % NSLOT == 2
  lax.fori_loop(0, NCHUNK // NSLOT, triple, 0)
  iteration(NCHUNK - 2, (NCHUNK - 2) % 3, (NCHUNK - 1) % 3, NCHUNK % 3)
  iteration(NCHUNK - 1, (NCHUNK - 1) % 3, NCHUNK % 3, (NCHUNK + 1) % 3)

  for sl in range(NSLOT):
    scat_wait(sl)

  if TAIL:
    tbase = ebase + NCHUNK * CHUNK
    pltpu.sync_copy(row_hbm.at[pl.ds(tbase, TAIL)], rowi_t)
    pltpu.sync_copy(col_hbm.at[pl.ds(tbase, TAIL)], coli_t)
    pltpu.sync_copy(val_hbm.at[pl.ds(tbase, TAIL)], val_t)
    pltpu.async_copy(h_hbm.at[coli_t], rows_t, sem_t).wait()
    _scale(rows_t, val_t, TAIL)
    pltpu.sync_copy(rows_t, acc_sh.at[rowi_t], add=True)

  plsc.subcore_barrier()
  pltpu.sync_copy(acc_sh.at[pl.ds(s * RPT, RPT)],
                  p_hbm.at[c, pl.ds(s * RPT, RPT)])

  @pl.when(s == NS - 1)
  def _():
    pltpu.sync_copy(acc_sh.at[pl.ds(NS * RPT, REM)],
                    p_hbm.at[c, pl.ds(NS * RPT, REM)])


@jax.jit
def _spmm(h, row, col, val):
  mesh = plsc.VectorSubcoreMesh(core_axis_name="c", subcore_axis_name="s")
  scratch = (
      [pltpu.VMEM((CHUNK, D_FEAT), jnp.float32)] * 3
      + [pltpu.VMEM((CHUNK,), jnp.int32)] * 3
      + [pltpu.VMEM((CHUNK,), jnp.int32)] * 3
      + [pltpu.VMEM((CHUNK,), jnp.float32)] * 3
      + [pltpu.VMEM((CHUNK,), jnp.int32)] * 3
      + [
          pltpu.VMEM((TAIL, D_FEAT), jnp.float32),
          pltpu.VMEM((TAIL,), jnp.int32),
          pltpu.VMEM((TAIL,), jnp.int32),
          pltpu.VMEM((TAIL,), jnp.float32),
          pltpu.VMEM_SHARED((N_NODES, D_FEAT), jnp.float32),
      ]
      + [pltpu.SemaphoreType.DMA] * 10
  )
  return pl.kernel(
      _spmm_body,
      out_type=jax.ShapeDtypeStruct((NC, N_NODES, D_FEAT), jnp.float32),
      mesh=mesh,
      scratch_types=scratch,
      name="gcn_spmm_sc",
  )(h, row, col, val)


def _dense_body(p_ref, w_ref, b_ref, o_ref):
  x = p_ref[0] + p_ref[1]
  y = jnp.dot(x, w_ref[...], preferred_element_type=jnp.float32) + b_ref[...]
  o_ref[...] = jnp.maximum(y, 0.0)


BLK = 1000


@jax.jit
def _dense(p, w, b):
  b2 = b.reshape(1, D_FEAT)
  grid = (N_NODES // BLK,)
  return pl.pallas_call(
      _dense_body,
      grid=grid,
      in_specs=[
          pl.BlockSpec((NC, BLK, D_FEAT), lambda i: (0, i, 0)),
          pl.BlockSpec((D_FEAT, D_FEAT), lambda i: (0, 0)),
          pl.BlockSpec((1, D_FEAT), lambda i: (0, 0)),
      ],
      out_specs=pl.BlockSpec((BLK, D_FEAT), lambda i: (i, 0)),
      out_shape=jax.ShapeDtypeStruct((N_NODES, D_FEAT), jnp.float32),
      name="gcn_dense_tc",
  )(p, w, b2)


def kernel(H, edge_index, edge_values, W0, b0, W1, b1):
  row = edge_index[0].astype(jnp.int32)
  col = edge_index[1].astype(jnp.int32)
  val = edge_values.astype(jnp.float32)
  p0 = _spmm(H, row, col, val)
  h1 = _dense(p0, W0, b0)
  p1 = _spmm(h1, row, col, val)
  h2 = _dense(p1, W1, b1)
  return h2
